# baseline (speedup 1.0000x reference)
import functools

import jax
import jax.numpy as jnp
from jax import lax
from jax.experimental import pallas as pl
from jax.experimental.pallas import tpu as pltpu
from jax.experimental.pallas import tpu_sc as plsc

N = 10000
E = 320000
D_IN = 128
H1 = 64
H2 = 32

NC = 2
NS = 16
NW = NC * NS

CHUNK = 128
CW = 84
EPAD = CW * NW * CHUNK
NPAD = 10112
RSLICE = NPAD // NS

_SC_PARAMS = pltpu.CompilerParams(
    needs_layout_passes=False, use_tc_tiling_on_sc=False)


def _tc_first(x_ref, w_ref, a_ref, h_ref, sd_ref):
    h = jnp.dot(x_ref[...], w_ref[...], preferred_element_type=jnp.float32)
    h_ref[...] = h
    sd_ref[...] = jnp.dot(h, a_ref[...], preferred_element_type=jnp.float32)


def _tc_mid(acc_ref, den_ref, w_ref, a_ref, h_ref, sd_ref):
    acc = acc_ref[0, :N, :] + acc_ref[1, :N, :]
    den = den_ref[0, :N, :] + den_ref[1, :N, :]
    x = jnp.maximum(acc / (den + 1e-16), 0.0)
    h = jnp.dot(x, w_ref[...], preferred_element_type=jnp.float32)
    h_ref[...] = h
    sd_ref[...] = jnp.dot(h, a_ref[...], preferred_element_type=jnp.float32)


def _tc_final(num_ref, den_ref, out_ref):
    num = num_ref[0, :N, :] + num_ref[1, :N, :]
    den = den_ref[0, :N, :] + den_ref[1, :N, :]
    out_ref[...] = jax.nn.sigmoid(num / (den + 1e-16))


def _score_group(sd_tab, src_all, dst_all, ch, g):
    s16 = src_all[ch, pl.ds(g * 16, 16)]
    d16 = dst_all[ch, pl.ds(g * 16, 16)]
    sv = plsc.load_gather(sd_tab, [s16 * 2])
    dv = plsc.load_gather(sd_tab, [d16 * 2 + 1])
    e = sv + dv
    e = jnp.where(e >= 0.0, e, e * 0.2)
    return s16, jnp.exp(e)


def _make_sc_edge_pass(F):
    mesh = plsc.VectorSubcoreMesh(core_axis_name="c", subcore_axis_name="s")
    sd_len = 2 * N + 32

    @functools.partial(
        pl.kernel,
        out_type=[
            jax.ShapeDtypeStruct((NC, NPAD, F), jnp.float32),
            jax.ShapeDtypeStruct((NC * NPAD,), jnp.float32),
        ],
        mesh=mesh,
        compiler_params=_SC_PARAMS,
        scratch_types=[
            pltpu.VMEM((sd_len,), jnp.float32),
            pltpu.VMEM((CW, CHUNK), jnp.int32),
            pltpu.VMEM((CW, CHUNK), jnp.int32),
            pltpu.VMEM((CW, CHUNK), jnp.float32),
            pltpu.VMEM((CHUNK, F), jnp.float32),
            pltpu.VMEM((CHUNK, F), jnp.float32),
            pltpu.VMEM((CHUNK, F), jnp.float32),
            pltpu.VMEM_SHARED((NPAD, F), jnp.float32),
            pltpu.VMEM_SHARED((NPAD,), jnp.float32),
            pltpu.SemaphoreType.DMA,
            pltpu.SemaphoreType.DMA,
            pltpu.SemaphoreType.DMA,
            pltpu.SemaphoreType.DMA,
            pltpu.SemaphoreType.DMA,
            pltpu.SemaphoreType.DMA,
            pltpu.SemaphoreType.DMA,
        ],
    )
    def edge_pass(src_hbm, dst_hbm, sd_hbm, h_hbm,
                  acc_out, den_out,
                  sd_tab, src_all, dst_all, w_all, rows0, rows1, rows2,
                  acc_sh, den_sh,
                  sg0, sg1, sg2, ss0, ss1, ss2, sden):
        rows = (rows0, rows1, rows2)
        sg = (sg0, sg1, sg2)
        ss = (ss0, ss1, ss2)
        c = lax.axis_index("c")
        s = lax.axis_index("s")
        w = s * NC + c

        pltpu.sync_copy(sd_hbm, sd_tab)
        pltpu.sync_copy(src_hbm.at[w], src_all)
        pltpu.sync_copy(dst_hbm.at[w], dst_all)

        zero16 = jnp.zeros((16,), jnp.float32)
        for ed in range(CHUNK):
            for t in range(F // 16):
                rows0[ed, pl.ds(t * 16, 16)] = zero16
        for g in range(CHUNK // 16):
            w_all[0, pl.ds(g * 16, 16)] = zero16
        r0 = s * RSLICE
        nfull, rem = divmod(RSLICE, CHUNK)
        for k in range(nfull + (1 if rem else 0)):
            ln = CHUNK if k < nfull else rem
            off = r0 + k * CHUNK
            pltpu.sync_copy(rows0.at[pl.ds(0, ln)], acc_sh.at[pl.ds(off, ln)])
            pltpu.sync_copy(w_all.at[0, pl.ds(0, ln)], den_sh.at[pl.ds(off, ln)])
        plsc.subcore_barrier()

        pltpu.async_copy(h_hbm.at[src_all.at[0]], rows0, sg[0])

        def score_chunk(ch, carry):
            for g in range(CHUNK // 16):
                _, wv = _score_group(sd_tab, src_all, dst_all, ch, g)
                w_all[ch, pl.ds(g * 16, 16)] = wv
            pltpu.async_copy(
                w_all.at[ch], den_sh.at[dst_all.at[ch]], sden, add=True)

            @pl.when(ch >= 8)
            def _():
                pltpu.make_async_copy(
                    sd_hbm.at[pl.ds(0, CHUNK)], w_all.at[0], sden).wait()
            return carry

        lax.fori_loop(0, CW, score_chunk, 0)

        def block(i, carry):
            for r in range(3):
                r1 = (r + 1) % 3
                ch = i * 3 + r

                @pl.when(ch >= 2)
                def _():
                    pltpu.make_async_copy(
                        h_hbm.at[pl.ds(0, CHUNK)], rows[r1], ss[r1]).wait()

                @pl.when(ch + 1 < CW)
                def _():
                    pltpu.async_copy(
                        h_hbm.at[src_all.at[ch + 1]], rows[r1], sg[r1])

                pltpu.make_async_copy(
                    h_hbm.at[pl.ds(0, CHUNK)], rows[r], sg[r]).wait()
                for ed in range(CHUNK):
                    ws = plsc.load_gather(
                        w_all, [jnp.full((16,), ch, jnp.int32),
                                jnp.full((16,), ed, jnp.int32)])
                    for t in range(F // 16):
                        rows[r][ed, pl.ds(t * 16, 16)] = (
                            rows[r][ed, pl.ds(t * 16, 16)] * ws)
                pltpu.async_copy(
                    rows[r], acc_sh.at[dst_all.at[ch]], ss[r], add=True)
            return carry

        lax.fori_loop(0, CW // 3, block, 0)

        pltpu.make_async_copy(
            h_hbm.at[pl.ds(0, CHUNK)], rows[(CW - 2) % 3], ss[(CW - 2) % 3]).wait()
        pltpu.make_async_copy(
            h_hbm.at[pl.ds(0, CHUNK)], rows[(CW - 1) % 3], ss[(CW - 1) % 3]).wait()

        def drain_den(i, carry):
            pltpu.make_async_copy(
                sd_hbm.at[pl.ds(0, CHUNK)], w_all.at[0], sden).wait()
            return carry

        lax.fori_loop(0, 8, drain_den, 0)
        plsc.subcore_barrier()

        for k in range(nfull + (1 if rem else 0)):
            ln = CHUNK if k < nfull else rem
            off = r0 + k * CHUNK
            pltpu.sync_copy(acc_sh.at[pl.ds(off, ln)], rows0.at[pl.ds(0, ln)])
            pltpu.sync_copy(rows0.at[pl.ds(0, ln)], acc_out.at[c, pl.ds(off, ln)])
            pltpu.sync_copy(den_sh.at[pl.ds(off, ln)], w_all.at[0, pl.ds(0, ln)])
            pltpu.sync_copy(w_all.at[0, pl.ds(0, ln)],
                            den_out.at[pl.ds(c * NPAD + off, ln)])

    return edge_pass


def _make_sc_edge_pass_scalar():
    mesh = plsc.VectorSubcoreMesh(core_axis_name="c", subcore_axis_name="s")
    sd_len = 2 * N + 32
    h_len = N + 16

    @functools.partial(
        pl.kernel,
        out_type=[
            jax.ShapeDtypeStruct((NC * NPAD,), jnp.float32),
            jax.ShapeDtypeStruct((NC * NPAD,), jnp.float32),
        ],
        mesh=mesh,
        compiler_params=_SC_PARAMS,
        scratch_types=[
            pltpu.VMEM((sd_len,), jnp.float32),
            pltpu.VMEM((h_len,), jnp.float32),
            pltpu.VMEM((CW, CHUNK), jnp.int32),
            pltpu.VMEM((CW, CHUNK), jnp.int32),
            pltpu.VMEM((CW, CHUNK), jnp.float32),
            pltpu.VMEM((CW, CHUNK), jnp.float32),
            pltpu.VMEM_SHARED((NPAD,), jnp.float32),
            pltpu.VMEM_SHARED((NPAD,), jnp.float32),
            pltpu.SemaphoreType.DMA,
            pltpu.SemaphoreType.DMA,
        ],
    )
    def edge_pass(src_hbm, dst_hbm, sd_hbm, h_hbm,
                  num_out, den_out,
                  sd_tab, h_tab, src_all, dst_all, w_all, num_all,
                  num_sh, den_sh, sden, snum):
        c = lax.axis_index("c")
        s = lax.axis_index("s")
        w = s * NC + c

        pltpu.sync_copy(sd_hbm, sd_tab)
        pltpu.sync_copy(h_hbm, h_tab)
        pltpu.sync_copy(src_hbm.at[w], src_all)
        pltpu.sync_copy(dst_hbm.at[w], dst_all)

        zero16 = jnp.zeros((16,), jnp.float32)
        for g in range(CHUNK // 16):
            w_all[0, pl.ds(g * 16, 16)] = zero16
        r0 = s * RSLICE
        nfull, rem = divmod(RSLICE, CHUNK)
        for k in range(nfull + (1 if rem else 0)):
            ln = CHUNK if k < nfull else rem
            off = r0 + k * CHUNK
            pltpu.sync_copy(w_all.at[0, pl.ds(0, ln)], num_sh.at[pl.ds(off, ln)])
            pltpu.sync_copy(w_all.at[0, pl.ds(0, ln)], den_sh.at[pl.ds(off, ln)])
        plsc.subcore_barrier()

        def score_chunk(ch, carry):
            for g in range(CHUNK // 16):
                s16, wv = _score_group(sd_tab, src_all, dst_all, ch, g)
                hv = plsc.load_gather(h_tab, [s16])
                w_all[ch, pl.ds(g * 16, 16)] = wv
                num_all[ch, pl.ds(g * 16, 16)] = wv * hv
            pltpu.async_copy(
                w_all.at[ch], den_sh.at[dst_all.at[ch]], sden, add=True)
            pltpu.async_copy(
                num_all.at[ch], num_sh.at[dst_all.at[ch]], snum, add=True)

            @pl.when(ch >= 8)
            def _():
                pltpu.make_async_copy(
                    sd_hbm.at[pl.ds(0, CHUNK)], w_all.at[0], sden).wait()
                pltpu.make_async_copy(
                    sd_hbm.at[pl.ds(0, CHUNK)], num_all.at[0], snum).wait()
            return carry

        lax.fori_loop(0, CW, score_chunk, 0)

        def drain(i, carry):
            pltpu.make_async_copy(
                sd_hbm.at[pl.ds(0, CHUNK)], w_all.at[0], sden).wait()
            pltpu.make_async_copy(
                sd_hbm.at[pl.ds(0, CHUNK)], num_all.at[0], snum).wait()
            return carry

        lax.fori_loop(0, 8, drain, 0)
        plsc.subcore_barrier()

        for k in range(nfull + (1 if rem else 0)):
            ln = CHUNK if k < nfull else rem
            off = r0 + k * CHUNK
            pltpu.sync_copy(num_sh.at[pl.ds(off, ln)], num_all.at[0, pl.ds(0, ln)])
            pltpu.sync_copy(num_all.at[0, pl.ds(0, ln)],
                            num_out.at[pl.ds(c * NPAD + off, ln)])
            pltpu.sync_copy(den_sh.at[pl.ds(off, ln)], w_all.at[0, pl.ds(0, ln)])
            pltpu.sync_copy(w_all.at[0, pl.ds(0, ln)],
                            den_out.at[pl.ds(c * NPAD + off, ln)])

    return edge_pass


_sc_pass_64 = _make_sc_edge_pass(H1)
_sc_pass_32 = _make_sc_edge_pass(H2)
_sc_pass_1 = _make_sc_edge_pass_scalar()


def kernel(feature, edge_index, W1, a1_src, a1_dst, W2, a2_src, a2_dst,
           W3, a3_src, a3_dst):
    f32 = jnp.float32
    src = edge_index[0]
    dst = edge_index[1]
    pad_e = EPAD - E
    src_p = jnp.concatenate([src, jnp.zeros((pad_e,), jnp.int32)])
    dst_p = jnp.concatenate([dst, jnp.full((pad_e,), N, jnp.int32)])
    src_p = src_p.reshape(NW, CW, CHUNK)
    dst_p = dst_p.reshape(NW, CW, CHUNK)

    A1 = jnp.stack([a1_src, a1_dst], axis=1)
    A2 = jnp.stack([a2_src, a2_dst], axis=1)
    A3 = jnp.stack([a3_src, a3_dst], axis=1)

    sd_pad = jnp.zeros((32,), f32)
    h_pad = jnp.zeros((16,), f32)

    h1, sd1 = pl.pallas_call(
        _tc_first,
        out_shape=[jax.ShapeDtypeStruct((N, H1), f32),
                   jax.ShapeDtypeStruct((N, 2), f32)],
    )(feature, W1, A1)

    acc1, den1 = _sc_pass_64(
        src_p, dst_p, jnp.concatenate([sd1.reshape(-1), sd_pad]), h1)

    h2, sd2 = pl.pallas_call(
        _tc_mid,
        out_shape=[jax.ShapeDtypeStruct((N, H2), f32),
                   jax.ShapeDtypeStruct((N, 2), f32)],
    )(acc1, den1.reshape(NC, NPAD, 1), W2, A2)

    acc2, den2 = _sc_pass_32(
        src_p, dst_p, jnp.concatenate([sd2.reshape(-1), sd_pad]), h2)

    h3, sd3 = pl.pallas_call(
        _tc_mid,
        out_shape=[jax.ShapeDtypeStruct((N, 1), f32),
                   jax.ShapeDtypeStruct((N, 2), f32)],
    )(acc2, den2.reshape(NC, NPAD, 1), W3, A3)

    num3, den3 = _sc_pass_1(
        src_p, dst_p, jnp.concatenate([sd3.reshape(-1), sd_pad]),
        jnp.concatenate([h3.reshape(-1), h_pad]))

    out = pl.pallas_call(
        _tc_final,
        out_shape=jax.ShapeDtypeStruct((N, 1), f32),
    )(num3.reshape(NC, NPAD, 1), den3.reshape(NC, NPAD, 1))
    return out


# speedup vs baseline: 1.0062x; 1.0062x over previous
import functools

import jax
import jax.numpy as jnp
from jax import lax
from jax.experimental import pallas as pl
from jax.experimental.pallas import tpu as pltpu
from jax.experimental.pallas import tpu_sc as plsc

N = 10000
E = 320000
D_IN = 128
H1 = 64
H2 = 32

NC = 2
NS = 16
NW = NC * NS

CHUNK = 128
CW = 84
EPAD = CW * NW * CHUNK
NPAD = 10112
RSLICE = NPAD // NS

_SC_PARAMS = pltpu.CompilerParams(
    needs_layout_passes=False, use_tc_tiling_on_sc=False)


def _tc_first(x_ref, w_ref, a_ref, h_ref, sd_ref):
    h = jnp.dot(x_ref[...], w_ref[...], preferred_element_type=jnp.float32)
    h_ref[...] = h
    sd_ref[...] = jnp.dot(h, a_ref[...], preferred_element_type=jnp.float32)


def _tc_mid(acc_ref, den_ref, w_ref, a_ref, h_ref, sd_ref):
    acc = acc_ref[0, :N, :] + acc_ref[1, :N, :]
    den = den_ref[0, :N, :] + den_ref[1, :N, :]
    x = jnp.maximum(acc / (den + 1e-16), 0.0)
    h = jnp.dot(x, w_ref[...], preferred_element_type=jnp.float32)
    h_ref[...] = h
    sd_ref[...] = jnp.dot(h, a_ref[...], preferred_element_type=jnp.float32)


def _tc_final(num_ref, den_ref, out_ref):
    num = num_ref[0, :N, :] + num_ref[1, :N, :]
    den = den_ref[0, :N, :] + den_ref[1, :N, :]
    out_ref[...] = jax.nn.sigmoid(num / (den + 1e-16))


def _score_group(sd_tab, src_all, dst_all, ch, g):
    s16 = src_all[ch, pl.ds(g * 16, 16)]
    d16 = dst_all[ch, pl.ds(g * 16, 16)]
    sv = plsc.load_gather(sd_tab, [s16 * 2])
    dv = plsc.load_gather(sd_tab, [d16 * 2 + 1])
    e = sv + dv
    e = jnp.where(e >= 0.0, e, e * 0.2)
    return s16, jnp.exp(e)


def _make_sc_edge_pass(F):
    mesh = plsc.VectorSubcoreMesh(core_axis_name="c", subcore_axis_name="s")
    sd_len = 2 * N + 32

    @functools.partial(
        pl.kernel,
        out_type=[
            jax.ShapeDtypeStruct((NC, NPAD, F), jnp.float32),
            jax.ShapeDtypeStruct((NC * NPAD,), jnp.float32),
        ],
        mesh=mesh,
        compiler_params=_SC_PARAMS,
        scratch_types=[
            pltpu.VMEM((sd_len,), jnp.float32),
            pltpu.VMEM((CW, CHUNK), jnp.int32),
            pltpu.VMEM((CW, CHUNK), jnp.int32),
            pltpu.VMEM((CW, CHUNK), jnp.float32),
            pltpu.VMEM((CHUNK, F), jnp.float32),
            pltpu.VMEM((CHUNK, F), jnp.float32),
            pltpu.VMEM((CHUNK, F), jnp.float32),
            pltpu.VMEM_SHARED((NPAD, F), jnp.float32),
            pltpu.VMEM_SHARED((NPAD,), jnp.float32),
            pltpu.SemaphoreType.DMA,
            pltpu.SemaphoreType.DMA,
            pltpu.SemaphoreType.DMA,
            pltpu.SemaphoreType.DMA,
            pltpu.SemaphoreType.DMA,
            pltpu.SemaphoreType.DMA,
            pltpu.SemaphoreType.DMA,
        ],
    )
    def edge_pass(src_hbm, dst_hbm, sd_hbm, h_hbm,
                  acc_out, den_out,
                  sd_tab, src_all, dst_all, w_all, rows0, rows1, rows2,
                  acc_sh, den_sh,
                  sg0, sg1, sg2, ss0, ss1, ss2, sden):
        rows = (rows0, rows1, rows2)
        sg = (sg0, sg1, sg2)
        ss = (ss0, ss1, ss2)
        c = lax.axis_index("c")
        s = lax.axis_index("s")
        w = s * NC + c

        pltpu.sync_copy(sd_hbm, sd_tab)
        pltpu.sync_copy(src_hbm.at[w], src_all)
        pltpu.sync_copy(dst_hbm.at[w], dst_all)

        zero16 = jnp.zeros((16,), jnp.float32)

        def zrow(ed, carry):
            for t in range(F // 16):
                rows0[ed, pl.ds(t * 16, 16)] = zero16
            return carry

        lax.fori_loop(0, CHUNK, zrow, 0)
        for g in range(CHUNK // 16):
            w_all[0, pl.ds(g * 16, 16)] = zero16
        r0 = s * RSLICE
        nfull, rem = divmod(RSLICE, CHUNK)
        for k in range(nfull + (1 if rem else 0)):
            ln = CHUNK if k < nfull else rem
            off = r0 + k * CHUNK
            pltpu.sync_copy(rows0.at[pl.ds(0, ln)], acc_sh.at[pl.ds(off, ln)])
            pltpu.sync_copy(w_all.at[0, pl.ds(0, ln)], den_sh.at[pl.ds(off, ln)])
        plsc.subcore_barrier()

        pltpu.async_copy(h_hbm.at[src_all.at[0]], rows0, sg[0])

        def score_chunk(ch, carry):
            for g in range(CHUNK // 16):
                _, wv = _score_group(sd_tab, src_all, dst_all, ch, g)
                w_all[ch, pl.ds(g * 16, 16)] = wv
            pltpu.async_copy(
                w_all.at[ch], den_sh.at[dst_all.at[ch]], sden, add=True)

            @pl.when(ch >= 8)
            def _():
                pltpu.make_async_copy(
                    sd_hbm.at[pl.ds(0, CHUNK)], w_all.at[0], sden).wait()
            return carry

        lax.fori_loop(0, CW, score_chunk, 0)

        def block(i, carry):
            for r in range(3):
                r1 = (r + 1) % 3
                ch = i * 3 + r

                @pl.when(ch >= 2)
                def _():
                    pltpu.make_async_copy(
                        h_hbm.at[pl.ds(0, CHUNK)], rows[r1], ss[r1]).wait()

                @pl.when(ch + 1 < CW)
                def _():
                    pltpu.async_copy(
                        h_hbm.at[src_all.at[ch + 1]], rows[r1], sg[r1])

                pltpu.make_async_copy(
                    h_hbm.at[pl.ds(0, CHUNK)], rows[r], sg[r]).wait()
                rr = rows[r]

                def scale_group(g, carry):
                    for eg in range(32):
                        ed = g * 32 + eg
                        ws = plsc.load_gather(
                            w_all, [jnp.full((16,), ch, jnp.int32),
                                    jnp.full((16,), ed, jnp.int32)])
                        for t in range(F // 16):
                            rr[ed, pl.ds(t * 16, 16)] = (
                                rr[ed, pl.ds(t * 16, 16)] * ws)
                    return carry

                lax.fori_loop(0, CHUNK // 32, scale_group, 0)
                pltpu.async_copy(
                    rows[r], acc_sh.at[dst_all.at[ch]], ss[r], add=True)
            return carry

        lax.fori_loop(0, CW // 3, block, 0)

        pltpu.make_async_copy(
            h_hbm.at[pl.ds(0, CHUNK)], rows[(CW - 2) % 3], ss[(CW - 2) % 3]).wait()
        pltpu.make_async_copy(
            h_hbm.at[pl.ds(0, CHUNK)], rows[(CW - 1) % 3], ss[(CW - 1) % 3]).wait()

        def drain_den(i, carry):
            pltpu.make_async_copy(
                sd_hbm.at[pl.ds(0, CHUNK)], w_all.at[0], sden).wait()
            return carry

        lax.fori_loop(0, 8, drain_den, 0)
        plsc.subcore_barrier()

        for k in range(nfull + (1 if rem else 0)):
            ln = CHUNK if k < nfull else rem
            off = r0 + k * CHUNK
            pltpu.sync_copy(acc_sh.at[pl.ds(off, ln)], rows0.at[pl.ds(0, ln)])
            pltpu.sync_copy(rows0.at[pl.ds(0, ln)], acc_out.at[c, pl.ds(off, ln)])
            pltpu.sync_copy(den_sh.at[pl.ds(off, ln)], w_all.at[0, pl.ds(0, ln)])
            pltpu.sync_copy(w_all.at[0, pl.ds(0, ln)],
                            den_out.at[pl.ds(c * NPAD + off, ln)])

    return edge_pass


def _make_sc_edge_pass_scalar():
    mesh = plsc.VectorSubcoreMesh(core_axis_name="c", subcore_axis_name="s")
    sd_len = 2 * N + 32
    h_len = N + 16

    @functools.partial(
        pl.kernel,
        out_type=[
            jax.ShapeDtypeStruct((NC * NPAD,), jnp.float32),
            jax.ShapeDtypeStruct((NC * NPAD,), jnp.float32),
        ],
        mesh=mesh,
        compiler_params=_SC_PARAMS,
        scratch_types=[
            pltpu.VMEM((sd_len,), jnp.float32),
            pltpu.VMEM((h_len,), jnp.float32),
            pltpu.VMEM((CW, CHUNK), jnp.int32),
            pltpu.VMEM((CW, CHUNK), jnp.int32),
            pltpu.VMEM((CW, CHUNK), jnp.float32),
            pltpu.VMEM((CW, CHUNK), jnp.float32),
            pltpu.VMEM_SHARED((NPAD,), jnp.float32),
            pltpu.VMEM_SHARED((NPAD,), jnp.float32),
            pltpu.SemaphoreType.DMA,
            pltpu.SemaphoreType.DMA,
        ],
    )
    def edge_pass(src_hbm, dst_hbm, sd_hbm, h_hbm,
                  num_out, den_out,
                  sd_tab, h_tab, src_all, dst_all, w_all, num_all,
                  num_sh, den_sh, sden, snum):
        c = lax.axis_index("c")
        s = lax.axis_index("s")
        w = s * NC + c

        pltpu.sync_copy(sd_hbm, sd_tab)
        pltpu.sync_copy(h_hbm, h_tab)
        pltpu.sync_copy(src_hbm.at[w], src_all)
        pltpu.sync_copy(dst_hbm.at[w], dst_all)

        zero16 = jnp.zeros((16,), jnp.float32)
        for g in range(CHUNK // 16):
            w_all[0, pl.ds(g * 16, 16)] = zero16
        r0 = s * RSLICE
        nfull, rem = divmod(RSLICE, CHUNK)
        for k in range(nfull + (1 if rem else 0)):
            ln = CHUNK if k < nfull else rem
            off = r0 + k * CHUNK
            pltpu.sync_copy(w_all.at[0, pl.ds(0, ln)], num_sh.at[pl.ds(off, ln)])
            pltpu.sync_copy(w_all.at[0, pl.ds(0, ln)], den_sh.at[pl.ds(off, ln)])
        plsc.subcore_barrier()

        def score_chunk(ch, carry):
            for g in range(CHUNK // 16):
                s16, wv = _score_group(sd_tab, src_all, dst_all, ch, g)
                hv = plsc.load_gather(h_tab, [s16])
                w_all[ch, pl.ds(g * 16, 16)] = wv
                num_all[ch, pl.ds(g * 16, 16)] = wv * hv
            pltpu.async_copy(
                w_all.at[ch], den_sh.at[dst_all.at[ch]], sden, add=True)
            pltpu.async_copy(
                num_all.at[ch], num_sh.at[dst_all.at[ch]], snum, add=True)

            @pl.when(ch >= 8)
            def _():
                pltpu.make_async_copy(
                    sd_hbm.at[pl.ds(0, CHUNK)], w_all.at[0], sden).wait()
                pltpu.make_async_copy(
                    sd_hbm.at[pl.ds(0, CHUNK)], num_all.at[0], snum).wait()
            return carry

        lax.fori_loop(0, CW, score_chunk, 0)

        def drain(i, carry):
            pltpu.make_async_copy(
                sd_hbm.at[pl.ds(0, CHUNK)], w_all.at[0], sden).wait()
            pltpu.make_async_copy(
                sd_hbm.at[pl.ds(0, CHUNK)], num_all.at[0], snum).wait()
            return carry

        lax.fori_loop(0, 8, drain, 0)
        plsc.subcore_barrier()

        for k in range(nfull + (1 if rem else 0)):
            ln = CHUNK if k < nfull else rem
            off = r0 + k * CHUNK
            pltpu.sync_copy(num_sh.at[pl.ds(off, ln)], num_all.at[0, pl.ds(0, ln)])
            pltpu.sync_copy(num_all.at[0, pl.ds(0, ln)],
                            num_out.at[pl.ds(c * NPAD + off, ln)])
            pltpu.sync_copy(den_sh.at[pl.ds(off, ln)], w_all.at[0, pl.ds(0, ln)])
            pltpu.sync_copy(w_all.at[0, pl.ds(0, ln)],
                            den_out.at[pl.ds(c * NPAD + off, ln)])

    return edge_pass


_sc_pass_64 = _make_sc_edge_pass(H1)
_sc_pass_32 = _make_sc_edge_pass(H2)
_sc_pass_1 = _make_sc_edge_pass_scalar()


def kernel(feature, edge_index, W1, a1_src, a1_dst, W2, a2_src, a2_dst,
           W3, a3_src, a3_dst):
    f32 = jnp.float32
    src = edge_index[0]
    dst = edge_index[1]
    pad_e = EPAD - E
    src_p = jnp.concatenate([src, jnp.zeros((pad_e,), jnp.int32)])
    dst_p = jnp.concatenate([dst, jnp.full((pad_e,), N, jnp.int32)])
    src_p = src_p.reshape(NW, CW, CHUNK)
    dst_p = dst_p.reshape(NW, CW, CHUNK)

    A1 = jnp.stack([a1_src, a1_dst], axis=1)
    A2 = jnp.stack([a2_src, a2_dst], axis=1)
    A3 = jnp.stack([a3_src, a3_dst], axis=1)

    sd_pad = jnp.zeros((32,), f32)
    h_pad = jnp.zeros((16,), f32)

    h1, sd1 = pl.pallas_call(
        _tc_first,
        out_shape=[jax.ShapeDtypeStruct((N, H1), f32),
                   jax.ShapeDtypeStruct((N, 2), f32)],
    )(feature, W1, A1)

    acc1, den1 = _sc_pass_64(
        src_p, dst_p, jnp.concatenate([sd1.reshape(-1), sd_pad]), h1)

    h2, sd2 = pl.pallas_call(
        _tc_mid,
        out_shape=[jax.ShapeDtypeStruct((N, H2), f32),
                   jax.ShapeDtypeStruct((N, 2), f32)],
    )(acc1, den1.reshape(NC, NPAD, 1), W2, A2)

    acc2, den2 = _sc_pass_32(
        src_p, dst_p, jnp.concatenate([sd2.reshape(-1), sd_pad]), h2)

    h3, sd3 = pl.pallas_call(
        _tc_mid,
        out_shape=[jax.ShapeDtypeStruct((N, 1), f32),
                   jax.ShapeDtypeStruct((N, 2), f32)],
    )(acc2, den2.reshape(NC, NPAD, 1), W3, A3)

    num3, den3 = _sc_pass_1(
        src_p, dst_p, jnp.concatenate([sd3.reshape(-1), sd_pad]),
        jnp.concatenate([h3.reshape(-1), h_pad]))

    out = pl.pallas_call(
        _tc_final,
        out_shape=jax.ShapeDtypeStruct((N, 1), f32),
    )(num3.reshape(NC, NPAD, 1), den3.reshape(NC, NPAD, 1))
    return out
